# Initial kernel scaffold; baseline (speedup 1.0000x reference)
import functools

import jax
import jax.numpy as jnp
from jax import lax
from jax.experimental import pallas as pl
from jax.experimental.pallas import tpu as pltpu
from jax.experimental.pallas import tpu_sc as plsc

N = 10000
E = 320000
CH = 128

NC = 2
NS = 16
NW = NC * NS

NP = 10240
EP = 323584
EPT = EP // NW
ECH = 128
NCHUNK = EPT // ECH
NPT = NP // NW
RPS = NP // NS
DCH = 1264

_MESH = plsc.VectorSubcoreMesh(core_axis_name="c", subcore_axis_name="s")


def _wid():
    return lax.axis_index("s") * NC + lax.axis_index("c")


def _mm_body(x_ref, w_ref, o_ref):
    o_ref[...] = jnp.dot(x_ref[...], w_ref[...],
                         preferred_element_type=jnp.float32)


def _matmul(xp, W):
    BM = 1280
    return pl.pallas_call(
        _mm_body,
        grid=(NP // BM,),
        in_specs=[pl.BlockSpec((BM, CH), lambda i: (i, 0)),
                  pl.BlockSpec((CH, CH), lambda i: (0, 0))],
        out_specs=pl.BlockSpec((BM, CH), lambda i: (i, 0)),
        out_shape=jax.ShapeDtypeStruct((NP, CH), jnp.float32),
    )(xp, W)


@functools.partial(
    pl.kernel,
    out_type=jax.ShapeDtypeStruct((NW, NP), jnp.float32),
    mesh=_MESH,
    scratch_types=[pltpu.VMEM((NP,), jnp.float32),
                   pltpu.VMEM((DCH,), jnp.int32)],
)
def _hist_kernel(dst_hbm, out_hbm, hist_v, idx_v):
    w = _wid()
    zeros = jnp.zeros((16,), jnp.float32)
    ones = jnp.ones((16,), jnp.float32)

    def zbody(i, c):
        hist_v[pl.ds(i * 16, 16)] = zeros
        return c
    lax.fori_loop(0, NP // 16, zbody, 0)

    def chunk(ci, c):
        pltpu.sync_copy(dst_hbm.at[pl.ds(w * EPT + ci * DCH, DCH)], idx_v)

        def grp(g, c2):
            idx = idx_v[pl.ds(g * 16, 16)]
            plsc.addupdate_scatter(hist_v, [idx], ones)
            return c2
        lax.fori_loop(0, DCH // 16, grp, 0)
        return c
    lax.fori_loop(0, EPT // DCH, chunk, 0)

    pltpu.sync_copy(hist_v, out_hbm.at[w])


@functools.partial(
    pl.kernel,
    out_type=(jax.ShapeDtypeStruct((NP, CH), jnp.float32),
              jax.ShapeDtypeStruct((NP,), jnp.float32)),
    mesh=_MESH,
    scratch_types=[pltpu.VMEM((NW, NPT), jnp.float32),
                   pltpu.VMEM((NPT,), jnp.float32),
                   pltpu.VMEM((NPT, CH), jnp.float32)],
)
def _dinv_y_kernel(hist_hbm, xw_hbm, y_hbm, dinv_hbm, h_v, dinv_v, xw_v):
    w = _wid()
    base = w * NPT
    pltpu.sync_copy(hist_hbm.at[:, pl.ds(base, NPT)], h_v)
    pltpu.sync_copy(xw_hbm.at[pl.ds(base, NPT)], xw_v)

    def gbody(g, c):
        deg = jnp.ones((16,), jnp.float32)
        for r in range(NW):
            deg = deg + h_v[r, pl.ds(g * 16, 16)]
        i = plsc.bitcast(deg, jnp.int32)
        i = jnp.int32(0x5F3759DF) - jnp.right_shift(i, jnp.int32(1))
        h = plsc.bitcast(i, jnp.float32)
        for _ in range(3):
            h = h * (1.5 - 0.5 * deg * h * h)
        dinv_v[pl.ds(g * 16, 16)] = h
        return c
    lax.fori_loop(0, NPT // 16, gbody, 0)

    def rbody(r, c):
        n = dinv_v[r]
        for k in range(8):
            xw_v[r, pl.ds(k * 16, 16)] = xw_v[r, pl.ds(k * 16, 16)] * n
        return c
    lax.fori_loop(0, NPT, rbody, 0)

    pltpu.sync_copy(xw_v, y_hbm.at[pl.ds(base, NPT)])
    pltpu.sync_copy(dinv_v, dinv_hbm.at[pl.ds(base, NPT)])


@functools.partial(
    pl.kernel,
    out_type=jax.ShapeDtypeStruct((NC, NP, CH), jnp.float32),
    mesh=_MESH,
    scratch_types=[
        pltpu.VMEM((ECH,), jnp.int32),
        pltpu.VMEM((ECH,), jnp.int32),
        pltpu.VMEM((ECH, CH), jnp.float32),
        pltpu.VMEM_SHARED((NP, CH), jnp.float32),
        pltpu.SemaphoreType.DMA,
    ],
)
def _edge_kernel(src_hbm, dst_hbm, y_hbm, part_hbm,
                 sidx_v, didx_v, rows_v, acc_sh, sem):
    c = lax.axis_index("c")
    s = lax.axis_index("s")
    w = s * NC + c
    zeros = jnp.zeros((16,), jnp.float32)

    def zbody(i, cc):
        for k in range(8):
            rows_v[i, pl.ds(k * 16, 16)] = zeros
        return cc
    lax.fori_loop(0, ECH, zbody, 0)
    for t in range(RPS // ECH):
        pltpu.sync_copy(rows_v, acc_sh.at[pl.ds(s * RPS + t * ECH, ECH)])
    plsc.subcore_barrier()

    def chunk(j, cc):
        off = w * EPT + j * ECH
        pltpu.sync_copy(src_hbm.at[pl.ds(off, ECH)], sidx_v)
        pltpu.sync_copy(dst_hbm.at[pl.ds(off, ECH)], didx_v)
        pltpu.async_copy(y_hbm.at[sidx_v], rows_v, sem).wait()
        pltpu.sync_copy(rows_v, acc_sh.at[didx_v], add=True)
        return cc
    lax.fori_loop(0, NCHUNK, chunk, 0)

    plsc.subcore_barrier()
    pltpu.sync_copy(acc_sh.at[pl.ds(s * RPS, RPS)],
                    part_hbm.at[c, pl.ds(s * RPS, RPS)])


HCH = NPT // 2


@functools.partial(
    pl.kernel,
    out_type=jax.ShapeDtypeStruct((NP, CH), jnp.float32),
    mesh=_MESH,
    scratch_types=[
        pltpu.VMEM((HCH, CH), jnp.float32),
        pltpu.VMEM((HCH, CH), jnp.float32),
        pltpu.VMEM((HCH, CH), jnp.float32),
        pltpu.VMEM((NPT,), jnp.float32),
        pltpu.VMEM((CH,), jnp.float32),
    ],
)
def _combine_kernel(part_hbm, y_hbm, dinv_hbm, b_hbm, out_hbm,
                    a_v, b_v, c_v, dinv_v, bias_v):
    w = _wid()
    base = w * NPT
    pltpu.sync_copy(dinv_hbm.at[pl.ds(base, NPT)], dinv_v)
    pltpu.sync_copy(b_hbm, bias_v)
    for half in range(2):
        hb = base + half * HCH
        pltpu.sync_copy(part_hbm.at[0, pl.ds(hb, HCH)], a_v)
        pltpu.sync_copy(part_hbm.at[1, pl.ds(hb, HCH)], b_v)
        pltpu.sync_copy(y_hbm.at[pl.ds(hb, HCH)], c_v)

        def rbody(r, cc):
            n = dinv_v[half * HCH + r]
            for k in range(8):
                sl = pl.ds(k * 16, 16)
                a_v[r, sl] = (a_v[r, sl] + b_v[r, sl] + c_v[r, sl]) * n \
                    + bias_v[sl]
            return cc
        lax.fori_loop(0, HCH, rbody, 0)
        pltpu.sync_copy(a_v, out_hbm.at[pl.ds(hb, HCH)])


def kernel(x, edge_index, W, b):
    x = x.astype(jnp.float32)
    src = edge_index[0].astype(jnp.int32)
    dst = edge_index[1].astype(jnp.int32)
    xp = jnp.zeros((NP, CH), jnp.float32).at[:N].set(x)
    pad = jnp.full((EP - E,), N, jnp.int32)
    srcp = jnp.concatenate([src, pad])
    dstp = jnp.concatenate([dst, pad])

    xw = _matmul(xp, W)
    hist = _hist_kernel(dstp)
    y, dinv = _dinv_y_kernel(hist, xw)
    part = _edge_kernel(srcp, dstp, y)
    out = _combine_kernel(part, y, dinv, b)
    return out[:N]


# speedup vs baseline: 15.0840x; 15.0840x over previous
import functools

import jax
import jax.numpy as jnp
from jax import lax
from jax.experimental import pallas as pl
from jax.experimental.pallas import tpu as pltpu
from jax.experimental.pallas import tpu_sc as plsc

N = 10000
E = 320000
CH = 128

NC = 2
NS = 16
NW = NC * NS

NP = 12288
EP = 327680
EPT = EP // NW
ECH = 128
NCHUNK = EPT // ECH
NPT = NP // NW
NPA = 10240
RPS = NPA // NS

_MESH = plsc.VectorSubcoreMesh(core_axis_name="c", subcore_axis_name="s")


def _wid():
    return lax.axis_index("s") * NC + lax.axis_index("c")


def _lane_gather(v, idx):
    return lax.gather(
        v, idx.reshape(16, 1),
        lax.GatherDimensionNumbers(offset_dims=(), collapsed_slice_dims=(0,),
                                   start_index_map=(0,)),
        (1,), mode=lax.GatherScatterMode.PROMISE_IN_BOUNDS)


def _bcast_lane(v, l):
    return _lane_gather(v, jnp.full((16,), l, jnp.int32))


def _mm_body(x_ref, w_ref, o_ref):
    o_ref[...] = jnp.dot(x_ref[...], w_ref[...],
                         preferred_element_type=jnp.float32)


def _matmul(xp, W):
    BM = 1536
    return pl.pallas_call(
        _mm_body,
        grid=(NP // BM,),
        in_specs=[pl.BlockSpec((BM, CH), lambda i: (i, 0)),
                  pl.BlockSpec((CH, CH), lambda i: (0, 0))],
        out_specs=pl.BlockSpec((BM, CH), lambda i: (i, 0)),
        out_shape=jax.ShapeDtypeStruct((NP, CH), jnp.float32),
    )(xp, W)


@functools.partial(
    pl.kernel,
    out_type=(jax.ShapeDtypeStruct((NW, NP), jnp.float32),
              jax.ShapeDtypeStruct((NW * NCHUNK, ECH), jnp.int32),
              jax.ShapeDtypeStruct((NW * NCHUNK, ECH), jnp.int32)),
    mesh=_MESH,
    compiler_params=pltpu.CompilerParams(needs_layout_passes=False),
    scratch_types=[pltpu.VMEM((NP,), jnp.float32),
                   pltpu.VMEM((NP,), jnp.int32),
                   pltpu.VMEM((EPT,), jnp.int32),
                   pltpu.VMEM((EPT,), jnp.int32),
                   pltpu.VMEM((NCHUNK, ECH), jnp.int32),
                   pltpu.VMEM((NCHUNK, ECH), jnp.int32),
                   pltpu.VMEM((16,), jnp.int32)],
)
def _hist_sort_kernel(src_hbm, dst_hbm, hist_hbm, src2_hbm, dst2_hbm,
                      hist_v, cnt_v, sraw_v, draw_v, sidx2_v, didx2_v,
                      tmp_v):
    w = _wid()
    zerosf = jnp.zeros((16,), jnp.float32)
    zerosi = jnp.zeros((16,), jnp.int32)
    onesf = jnp.ones((16,), jnp.float32)
    onesi = jnp.ones((16,), jnp.int32)
    iota = lax.iota(jnp.int32, 16)
    prev = jnp.maximum(iota - 1, 0)

    pltpu.sync_copy(src_hbm.at[pl.ds(w * EPT, EPT)], sraw_v)
    pltpu.sync_copy(dst_hbm.at[pl.ds(w * EPT, EPT)], draw_v)

    def zbody(i, c):
        hist_v[pl.ds(i * 16, 16)] = zerosf
        cnt_v[pl.ds(i * 16, 16)] = zerosi
        return c
    lax.fori_loop(0, NP // 16, zbody, 0)

    def ha(g, c):
        dstv = draw_v[pl.ds(g * 16, 16)]
        plsc.addupdate_scatter(hist_v, [dstv], onesf)
        srcv = sraw_v[pl.ds(g * 16, 16)]
        plsc.addupdate_scatter(cnt_v, [srcv], onesi)
        return c
    lax.fori_loop(0, EPT // 16, ha, 0)

    def pf(g, carry):
        v = cnt_v[pl.ds(g * 16, 16)]
        inc = plsc.cumsum(v)
        cnt_v[pl.ds(g * 16, 16)] = carry + inc - v
        return carry + _bcast_lane(inc, 15)
    lax.fori_loop(0, NP // 16, pf, zerosi)

    def hb(g, c):
        srcv = sraw_v[pl.ds(g * 16, 16)]
        dstv = draw_v[pl.ds(g * 16, 16)]
        base = plsc.load_gather(cnt_v, [srcv])
        sb, p = plsc.sort_key_val(srcv, iota)
        run = jnp.logical_or(sb != _lane_gather(sb, prev), iota == 0)
        start = plsc.cummax(jnp.where(run, iota, zerosi))
        plsc.store_scatter(tmp_v, [p], iota - start)
        rank = tmp_v[...]
        pos = base + rank
        plsc.addupdate_scatter(cnt_v, [srcv], onesi)
        r_hi = lax.shift_right_logical(pos, jnp.int32(7))
        r_lo = jnp.bitwise_and(pos, jnp.int32(127))
        plsc.store_scatter(sidx2_v, [r_hi, r_lo], srcv)
        plsc.store_scatter(didx2_v, [r_hi, r_lo], dstv)
        return c
    lax.fori_loop(0, EPT // 16, hb, 0)

    pltpu.sync_copy(hist_v, hist_hbm.at[w])
    pltpu.sync_copy(sidx2_v, src2_hbm.at[pl.ds(w * NCHUNK, NCHUNK)])
    pltpu.sync_copy(didx2_v, dst2_hbm.at[pl.ds(w * NCHUNK, NCHUNK)])


@functools.partial(
    pl.kernel,
    out_type=(jax.ShapeDtypeStruct((NP, CH), jnp.float32),
              jax.ShapeDtypeStruct((NP,), jnp.float32)),
    mesh=_MESH,
    compiler_params=pltpu.CompilerParams(needs_layout_passes=False),
    scratch_types=[pltpu.VMEM((NW, NPT), jnp.float32),
                   pltpu.VMEM((NPT,), jnp.float32),
                   pltpu.VMEM((NPT, CH), jnp.float32)],
)
def _dinv_y_kernel(hist_hbm, xw_hbm, y_hbm, dinv_hbm, h_v, dinv_v, xw_v):
    w = _wid()
    base = w * NPT
    pltpu.sync_copy(hist_hbm.at[:, pl.ds(base, NPT)], h_v)
    pltpu.sync_copy(xw_hbm.at[pl.ds(base, NPT)], xw_v)

    def gbody(g, c):
        deg = jnp.ones((16,), jnp.float32)
        for r in range(NW):
            deg = deg + h_v[r, pl.ds(g * 16, 16)]
        i = plsc.bitcast(deg, jnp.int32)
        i = jnp.int32(0x5F3759DF) - jnp.right_shift(i, jnp.int32(1))
        h = plsc.bitcast(i, jnp.float32)
        for _ in range(3):
            h = h * (1.5 - 0.5 * deg * h * h)
        dinv_v[pl.ds(g * 16, 16)] = h
        return c
    lax.fori_loop(0, NPT // 16, gbody, 0)

    def rbody(g, c):
        dv = dinv_v[pl.ds(g * 16, 16)]
        for l in range(16):
            n16 = _bcast_lane(dv, l)
            r = g * 16 + l
            for k in range(8):
                xw_v[r, pl.ds(k * 16, 16)] = xw_v[r, pl.ds(k * 16, 16)] * n16
        return c
    lax.fori_loop(0, NPT // 16, rbody, 0)

    pltpu.sync_copy(xw_v, y_hbm.at[pl.ds(base, NPT)])
    pltpu.sync_copy(dinv_v, dinv_hbm.at[pl.ds(base, NPT)])


@functools.partial(
    pl.kernel,
    out_type=jax.ShapeDtypeStruct((NC, NP, CH), jnp.float32),
    mesh=_MESH,
    compiler_params=pltpu.CompilerParams(needs_layout_passes=False),
    scratch_types=[
        pltpu.VMEM((NCHUNK, ECH), jnp.int32),
        pltpu.VMEM((ECH,), jnp.int32),
        pltpu.VMEM((ECH,), jnp.int32),
        pltpu.VMEM((ECH, CH), jnp.float32),
        pltpu.VMEM((ECH, CH), jnp.float32),
        pltpu.VMEM_SHARED((NPA, CH), jnp.float32),
        pltpu.SemaphoreType.DMA,
        pltpu.SemaphoreType.DMA,
        pltpu.SemaphoreType.DMA,
        pltpu.SemaphoreType.DMA,
    ],
)
def _edge_kernel(src_hbm, dst_hbm, y_hbm, part_hbm,
                 sidx_v, didx0_v, didx1_v, rows0_v, rows1_v, acc_sh,
                 g0, g1, d0, d1):
    c = lax.axis_index("c")
    s = lax.axis_index("s")
    w = s * NC + c
    zeros = jnp.zeros((16,), jnp.float32)

    pltpu.sync_copy(src_hbm.at[pl.ds(w * NCHUNK, NCHUNK)], sidx_v)

    def zbody(i, cc):
        for k in range(8):
            rows0_v[i, pl.ds(k * 16, 16)] = zeros
        return cc
    lax.fori_loop(0, ECH, zbody, 0)
    for t in range(RPS // ECH):
        pltpu.sync_copy(rows0_v, acc_sh.at[pl.ds(s * RPS + t * ECH, ECH)])
    plsc.subcore_barrier()

    ebase = w * EPT
    pltpu.sync_copy(dst_hbm.at[pl.ds(ebase, ECH)], didx0_v)
    pltpu.sync_copy(dst_hbm.at[pl.ds(ebase + ECH, ECH)], didx1_v)
    pltpu.async_copy(y_hbm.at[sidx_v.at[0]], rows0_v, g0)

    def chunk(g, cc):
        j0 = 2 * g
        pltpu.async_copy(y_hbm.at[sidx_v.at[j0 + 1]], rows1_v, g1)
        pltpu.make_async_copy(y_hbm.at[sidx_v.at[j0]], rows0_v, g0,
                              ).wait()

        @pl.when(g > 0)
        def _():
            pltpu.make_async_copy(dst_hbm.at[pl.ds(ebase + j0 * ECH, ECH)],
                                  didx0_v, d0).wait()
        pltpu.sync_copy(rows0_v, acc_sh.at[didx0_v], add=True)

        @pl.when(g < NCHUNK // 2 - 1)
        def _():
            pltpu.async_copy(dst_hbm.at[pl.ds(ebase + (j0 + 2) * ECH, ECH)],
                             didx0_v, d0)
            pltpu.async_copy(y_hbm.at[sidx_v.at[j0 + 2]], rows0_v, g0)
        pltpu.make_async_copy(y_hbm.at[sidx_v.at[j0 + 1]], rows1_v, g1,
                              ).wait()

        @pl.when(g > 0)
        def _():
            pltpu.make_async_copy(dst_hbm.at[pl.ds(ebase + (j0 + 1) * ECH,
                                                   ECH)], didx1_v, d1).wait()
        pltpu.sync_copy(rows1_v, acc_sh.at[didx1_v], add=True)

        @pl.when(g < NCHUNK // 2 - 1)
        def _():
            pltpu.async_copy(dst_hbm.at[pl.ds(ebase + (j0 + 3) * ECH, ECH)],
                             didx1_v, d1)
        return cc
    lax.fori_loop(0, NCHUNK // 2, chunk, 0)

    plsc.subcore_barrier()
    pltpu.sync_copy(acc_sh.at[pl.ds(s * RPS, RPS)],
                    part_hbm.at[c, pl.ds(s * RPS, RPS)])


HCH = NPT // 2


@functools.partial(
    pl.kernel,
    out_type=jax.ShapeDtypeStruct((NP, CH), jnp.float32),
    mesh=_MESH,
    compiler_params=pltpu.CompilerParams(needs_layout_passes=False),
    scratch_types=[
        pltpu.VMEM((HCH, CH), jnp.float32),
        pltpu.VMEM((HCH, CH), jnp.float32),
        pltpu.VMEM((HCH, CH), jnp.float32),
        pltpu.VMEM((NPT,), jnp.float32),
        pltpu.VMEM((CH,), jnp.float32),
    ],
)
def _combine_kernel(part_hbm, y_hbm, dinv_hbm, b_hbm, out_hbm,
                    a_v, b_v, c_v, dinv_v, bias_v):
    w = _wid()
    base = w * NPT
    pltpu.sync_copy(dinv_hbm.at[pl.ds(base, NPT)], dinv_v)
    pltpu.sync_copy(b_hbm, bias_v)
    bvecs = [bias_v[pl.ds(k * 16, 16)] for k in range(8)]
    for half in range(2):
        hb = base + half * HCH
        pltpu.sync_copy(part_hbm.at[0, pl.ds(hb, HCH)], a_v)
        pltpu.sync_copy(part_hbm.at[1, pl.ds(hb, HCH)], b_v)
        pltpu.sync_copy(y_hbm.at[pl.ds(hb, HCH)], c_v)

        def rbody(g, cc):
            dv = dinv_v[pl.ds(half * HCH + g * 16, 16)]
            for l in range(16):
                n16 = _bcast_lane(dv, l)
                r = g * 16 + l
                for k in range(8):
                    sl = pl.ds(k * 16, 16)
                    a_v[r, sl] = (a_v[r, sl] + b_v[r, sl] + c_v[r, sl]) \
                        * n16 + bvecs[k]
            return cc
        lax.fori_loop(0, HCH // 16, rbody, 0)
        pltpu.sync_copy(a_v, out_hbm.at[pl.ds(hb, HCH)])


def kernel(x, edge_index, W, b):
    x = x.astype(jnp.float32)
    src = edge_index[0].astype(jnp.int32)
    dst = edge_index[1].astype(jnp.int32)
    xp = jnp.zeros((NP, CH), jnp.float32).at[:N].set(x)
    pad = jnp.full((EP - E,), N, jnp.int32)
    srcp = jnp.concatenate([src, pad])
    dstp = jnp.concatenate([dst, pad])

    xw = _matmul(xp, W)
    hist, src2, dst2 = _hist_sort_kernel(srcp, dstp)
    y, dinv = _dinv_y_kernel(hist, xw)
    part = _edge_kernel(src2, dst2.reshape(EP), y)
    out = _combine_kernel(part, y, dinv, b)
    return out[:N]


# speedup vs baseline: 15.2340x; 1.0099x over previous
import functools

import jax
import jax.numpy as jnp
from jax import lax
from jax.experimental import pallas as pl
from jax.experimental.pallas import tpu as pltpu
from jax.experimental.pallas import tpu_sc as plsc

N = 10000
E = 320000
CH = 128

NC = 2
NS = 16
NW = NC * NS

NP = 12288
EP = 327680
EPT = EP // NW
ECH = 128
NCHUNK = EPT // ECH
NPT = NP // NW
NPA = 10112
RPS = NPA // NS
NBUF = 3

_MESH = plsc.VectorSubcoreMesh(core_axis_name="c", subcore_axis_name="s")


def _wid():
    return lax.axis_index("s") * NC + lax.axis_index("c")


def _lane_gather(v, idx):
    return lax.gather(
        v, idx.reshape(16, 1),
        lax.GatherDimensionNumbers(offset_dims=(), collapsed_slice_dims=(0,),
                                   start_index_map=(0,)),
        (1,), mode=lax.GatherScatterMode.PROMISE_IN_BOUNDS)


def _bcast_lane(v, l):
    return _lane_gather(v, jnp.full((16,), l, jnp.int32))


def _mm_body(x_ref, w_ref, o_ref):
    o_ref[...] = jnp.dot(x_ref[...], w_ref[...],
                         preferred_element_type=jnp.float32)


def _matmul(xp, W):
    BM = 1536
    return pl.pallas_call(
        _mm_body,
        grid=(NP // BM,),
        in_specs=[pl.BlockSpec((BM, CH), lambda i: (i, 0)),
                  pl.BlockSpec((CH, CH), lambda i: (0, 0))],
        out_specs=pl.BlockSpec((BM, CH), lambda i: (i, 0)),
        out_shape=jax.ShapeDtypeStruct((NP, CH), jnp.float32),
    )(xp, W)


@functools.partial(
    pl.kernel,
    out_type=(jax.ShapeDtypeStruct((NW, NP), jnp.float32),
              jax.ShapeDtypeStruct((NW * NCHUNK, ECH), jnp.int32),
              jax.ShapeDtypeStruct((NW * NCHUNK, ECH), jnp.int32)),
    mesh=_MESH,
    compiler_params=pltpu.CompilerParams(needs_layout_passes=False),
    scratch_types=[pltpu.VMEM((NP,), jnp.float32),
                   pltpu.VMEM((NP,), jnp.int32),
                   pltpu.VMEM((EPT,), jnp.int32),
                   pltpu.VMEM((EPT,), jnp.int32),
                   pltpu.VMEM((NCHUNK, ECH), jnp.int32),
                   pltpu.VMEM((NCHUNK, ECH), jnp.int32),
                   pltpu.VMEM((16,), jnp.int32)],
)
def _hist_sort_kernel(src_hbm, dst_hbm, hist_hbm, src2_hbm, dst2_hbm,
                      hist_v, cnt_v, sraw_v, draw_v, sidx2_v, didx2_v,
                      tmp_v):
    w = _wid()
    zerosf = jnp.zeros((16,), jnp.float32)
    zerosi = jnp.zeros((16,), jnp.int32)
    onesf = jnp.ones((16,), jnp.float32)
    onesi = jnp.ones((16,), jnp.int32)
    iota = lax.iota(jnp.int32, 16)
    prev = jnp.maximum(iota - 1, 0)

    pltpu.sync_copy(src_hbm.at[pl.ds(w * EPT, EPT)], sraw_v)
    pltpu.sync_copy(dst_hbm.at[pl.ds(w * EPT, EPT)], draw_v)

    def zbody(i, c):
        hist_v[pl.ds(i * 16, 16)] = zerosf
        cnt_v[pl.ds(i * 16, 16)] = zerosi
        return c
    lax.fori_loop(0, NP // 16, zbody, 0)

    def ha(g, c):
        dstv = draw_v[pl.ds(g * 16, 16)]
        plsc.addupdate_scatter(hist_v, [dstv], onesf)
        srcv = sraw_v[pl.ds(g * 16, 16)]
        plsc.addupdate_scatter(cnt_v, [srcv], onesi)
        return c
    lax.fori_loop(0, EPT // 16, ha, 0)

    def pf(g, carry):
        v = cnt_v[pl.ds(g * 16, 16)]
        inc = plsc.cumsum(v)
        cnt_v[pl.ds(g * 16, 16)] = carry + inc - v
        return carry + _bcast_lane(inc, 15)
    lax.fori_loop(0, NP // 16, pf, zerosi)

    def hb(g, c):
        srcv = sraw_v[pl.ds(g * 16, 16)]
        dstv = draw_v[pl.ds(g * 16, 16)]
        base = plsc.load_gather(cnt_v, [srcv])
        sb, p = plsc.sort_key_val(srcv, iota)
        run = jnp.logical_or(sb != _lane_gather(sb, prev), iota == 0)
        start = plsc.cummax(jnp.where(run, iota, zerosi))
        plsc.store_scatter(tmp_v, [p], iota - start)
        rank = tmp_v[...]
        pos = base + rank
        plsc.addupdate_scatter(cnt_v, [srcv], onesi)
        r_hi = lax.shift_right_logical(pos, jnp.int32(7))
        r_lo = jnp.bitwise_and(pos, jnp.int32(127))
        plsc.store_scatter(sidx2_v, [r_hi, r_lo], srcv)
        plsc.store_scatter(didx2_v, [r_hi, r_lo], dstv)
        return c
    lax.fori_loop(0, EPT // 16, hb, 0)

    pltpu.sync_copy(hist_v, hist_hbm.at[w])
    pltpu.sync_copy(sidx2_v, src2_hbm.at[pl.ds(w * NCHUNK, NCHUNK)])
    pltpu.sync_copy(didx2_v, dst2_hbm.at[pl.ds(w * NCHUNK, NCHUNK)])


@functools.partial(
    pl.kernel,
    out_type=(jax.ShapeDtypeStruct((NP, CH), jnp.float32),
              jax.ShapeDtypeStruct((NP,), jnp.float32)),
    mesh=_MESH,
    compiler_params=pltpu.CompilerParams(needs_layout_passes=False),
    scratch_types=[pltpu.VMEM((NW, NPT), jnp.float32),
                   pltpu.VMEM((NPT,), jnp.float32),
                   pltpu.VMEM((NPT, CH), jnp.float32)],
)
def _dinv_y_kernel(hist_hbm, xw_hbm, y_hbm, dinv_hbm, h_v, dinv_v, xw_v):
    w = _wid()
    base = w * NPT
    pltpu.sync_copy(hist_hbm.at[:, pl.ds(base, NPT)], h_v)
    pltpu.sync_copy(xw_hbm.at[pl.ds(base, NPT)], xw_v)

    def gbody(g, c):
        deg = jnp.ones((16,), jnp.float32)
        for r in range(NW):
            deg = deg + h_v[r, pl.ds(g * 16, 16)]
        i = plsc.bitcast(deg, jnp.int32)
        i = jnp.int32(0x5F3759DF) - jnp.right_shift(i, jnp.int32(1))
        h = plsc.bitcast(i, jnp.float32)
        for _ in range(3):
            h = h * (1.5 - 0.5 * deg * h * h)
        dinv_v[pl.ds(g * 16, 16)] = h
        return c
    lax.fori_loop(0, NPT // 16, gbody, 0)

    def rbody(g, c):
        dv = dinv_v[pl.ds(g * 16, 16)]
        for l in range(16):
            n16 = _bcast_lane(dv, l)
            r = g * 16 + l
            for k in range(8):
                xw_v[r, pl.ds(k * 16, 16)] = xw_v[r, pl.ds(k * 16, 16)] * n16
        return c
    lax.fori_loop(0, NPT // 16, rbody, 0)

    pltpu.sync_copy(xw_v, y_hbm.at[pl.ds(base, NPT)])
    pltpu.sync_copy(dinv_v, dinv_hbm.at[pl.ds(base, NPT)])


@functools.partial(
    pl.kernel,
    out_type=jax.ShapeDtypeStruct((NC, NP, CH), jnp.float32),
    mesh=_MESH,
    compiler_params=pltpu.CompilerParams(needs_layout_passes=False),
    scratch_types=[
        pltpu.VMEM((ECH,), jnp.int32),
        pltpu.VMEM((ECH,), jnp.int32),
        pltpu.VMEM((ECH,), jnp.int32),
        pltpu.VMEM((ECH,), jnp.int32),
        pltpu.VMEM((ECH,), jnp.int32),
        pltpu.VMEM((ECH,), jnp.int32),
        pltpu.VMEM((ECH,), jnp.int32),
        pltpu.VMEM((ECH, CH), jnp.float32),
        pltpu.VMEM((ECH, CH), jnp.float32),
        pltpu.VMEM((ECH, CH), jnp.float32),
        pltpu.VMEM_SHARED((NPA, CH), jnp.float32),
        pltpu.SemaphoreType.DMA,
        pltpu.SemaphoreType.DMA,
        pltpu.SemaphoreType.DMA,
    ],
)
def _edge_kernel(src_hbm, dst_hbm, y_hbm, part_hbm,
                 s0_v, s1_v, s2_v, s3_v, d0_v, d1_v, d2_v,
                 r0_v, r1_v, r2_v, acc_sh, gsem, xsem, dsem):
    c = lax.axis_index("c")
    s = lax.axis_index("s")
    w = s * NC + c
    zeros = jnp.zeros((16,), jnp.float32)
    sbufs = [s0_v, s1_v, s2_v, s3_v]
    dbufs = [d0_v, d1_v, d2_v]
    rbufs = [r0_v, r1_v, r2_v]

    def zbody(i, cc):
        for k in range(8):
            r0_v[i, pl.ds(k * 16, 16)] = zeros
        return cc
    lax.fori_loop(0, ECH, zbody, 0)
    for t in range(RPS // ECH):
        pltpu.sync_copy(r0_v, acc_sh.at[pl.ds(s * RPS + t * ECH, ECH)])
    pltpu.sync_copy(r0_v.at[pl.ds(0, RPS % ECH)],
                    acc_sh.at[pl.ds(s * RPS + (RPS // ECH) * ECH,
                                    RPS % ECH)])
    plsc.subcore_barrier()

    ebase = w * EPT

    def fire_sidx(j, sb):
        @pl.when(j < NCHUNK)
        def _():
            pltpu.async_copy(src_hbm.at[pl.ds(ebase + j * ECH, ECH)],
                             sbufs[sb], xsem)

    def fire_didx(j, db):
        @pl.when(j < NCHUNK)
        def _():
            pltpu.async_copy(dst_hbm.at[pl.ds(ebase + j * ECH, ECH)],
                             dbufs[db], dsem)

    def fire_gather(j, sb, rb):
        @pl.when(j < NCHUNK)
        def _():
            pltpu.make_async_copy(src_hbm.at[pl.ds(0, ECH)],
                                  sbufs[sb], xsem).wait()
            pltpu.async_copy(y_hbm.at[sbufs[sb]], rbufs[rb], gsem)

    def consume(j, i):
        rb, sb = i % NBUF, i % 4
        pltpu.make_async_copy(y_hbm.at[sbufs[sb]], rbufs[rb], gsem).wait()
        pltpu.make_async_copy(src_hbm.at[pl.ds(0, ECH)], dbufs[rb],
                              dsem).wait()
        pltpu.sync_copy(rbufs[rb], acc_sh.at[dbufs[rb]], add=True)
        fire_sidx(j + 4, (i + 4) % 4)
        fire_didx(j + NBUF, rb)
        fire_gather(j + NBUF, (i + NBUF) % 4, rb)

    for j in range(4):
        fire_sidx(j, j)
    for j in range(NBUF):
        fire_didx(j, j)
        fire_gather(j, j, j)

    TURN = 12

    def chunk(g, cc):
        j0 = TURN * g
        for i in range(TURN):
            consume(j0 + i, i)
        return cc
    nfull = NCHUNK // TURN
    lax.fori_loop(0, nfull, chunk, 0)
    for t in range(NCHUNK - nfull * TURN):
        consume(nfull * TURN + t, t)

    plsc.subcore_barrier()
    pltpu.sync_copy(acc_sh.at[pl.ds(s * RPS, RPS)],
                    part_hbm.at[c, pl.ds(s * RPS, RPS)])


HCH = NPT // 2


@functools.partial(
    pl.kernel,
    out_type=jax.ShapeDtypeStruct((NP, CH), jnp.float32),
    mesh=_MESH,
    compiler_params=pltpu.CompilerParams(needs_layout_passes=False),
    scratch_types=[
        pltpu.VMEM((HCH, CH), jnp.float32),
        pltpu.VMEM((HCH, CH), jnp.float32),
        pltpu.VMEM((HCH, CH), jnp.float32),
        pltpu.VMEM((NPT,), jnp.float32),
        pltpu.VMEM((CH,), jnp.float32),
    ],
)
def _combine_kernel(part_hbm, y_hbm, dinv_hbm, b_hbm, out_hbm,
                    a_v, b_v, c_v, dinv_v, bias_v):
    w = _wid()
    base = w * NPT
    pltpu.sync_copy(dinv_hbm.at[pl.ds(base, NPT)], dinv_v)
    pltpu.sync_copy(b_hbm, bias_v)
    bvecs = [bias_v[pl.ds(k * 16, 16)] for k in range(8)]
    for half in range(2):
        hb = base + half * HCH
        pltpu.sync_copy(part_hbm.at[0, pl.ds(hb, HCH)], a_v)
        pltpu.sync_copy(part_hbm.at[1, pl.ds(hb, HCH)], b_v)
        pltpu.sync_copy(y_hbm.at[pl.ds(hb, HCH)], c_v)

        def rbody(g, cc):
            dv = dinv_v[pl.ds(half * HCH + g * 16, 16)]
            for l in range(16):
                n16 = _bcast_lane(dv, l)
                r = g * 16 + l
                for k in range(8):
                    sl = pl.ds(k * 16, 16)
                    a_v[r, sl] = (a_v[r, sl] + b_v[r, sl] + c_v[r, sl]) \
                        * n16 + bvecs[k]
            return cc
        lax.fori_loop(0, HCH // 16, rbody, 0)
        pltpu.sync_copy(a_v, out_hbm.at[pl.ds(hb, HCH)])


def kernel(x, edge_index, W, b):
    x = x.astype(jnp.float32)
    src = edge_index[0].astype(jnp.int32)
    dst = edge_index[1].astype(jnp.int32)
    xp = jnp.zeros((NP, CH), jnp.float32).at[:N].set(x)
    pad = jnp.full((EP - E,), N, jnp.int32)
    srcp = jnp.concatenate([src, pad])
    dstp = jnp.concatenate([dst, pad])

    xw = _matmul(xp, W)
    hist, src2, dst2 = _hist_sort_kernel(srcp, dstp)
    y, dinv = _dinv_y_kernel(hist, xw)
    part = _edge_kernel(src2.reshape(EP), dst2.reshape(EP), y)
    out = _combine_kernel(part, y, dinv, b)
    return out[:N]


# speedup vs baseline: 16.6635x; 1.0938x over previous
import functools

import jax
import jax.numpy as jnp
from jax import lax
from jax.experimental import pallas as pl
from jax.experimental.pallas import tpu as pltpu
from jax.experimental.pallas import tpu_sc as plsc

N = 10000
E = 320000
CH = 128

NC = 2
NS = 16
NW = NC * NS

NP = 12288
EP = 323584
EPT = EP // NW
ECH = 128
NCHUNK = EPT // ECH
NPT = NP // NW
RPS = NP // NS

_MESH = plsc.VectorSubcoreMesh(core_axis_name="c", subcore_axis_name="s")


def _wid():
    return lax.axis_index("s") * NC + lax.axis_index("c")


def _bcast_lane(v, l):
    idx = jnp.full((16, 1), l, jnp.int32)
    return lax.gather(
        v, idx,
        lax.GatherDimensionNumbers(offset_dims=(), collapsed_slice_dims=(0,),
                                   start_index_map=(0,)),
        (1,), mode=lax.GatherScatterMode.PROMISE_IN_BOUNDS)


def _mm_body(x_ref, w_ref, o_ref):
    o_ref[...] = jnp.dot(x_ref[...], w_ref[...],
                         preferred_element_type=jnp.float32)


def _matmul(xp, W):
    BM = 1536
    return pl.pallas_call(
        _mm_body,
        grid=(NP // BM,),
        in_specs=[pl.BlockSpec((BM, CH), lambda i: (i, 0)),
                  pl.BlockSpec((CH, CH), lambda i: (0, 0))],
        out_specs=pl.BlockSpec((BM, CH), lambda i: (i, 0)),
        out_shape=jax.ShapeDtypeStruct((NP, CH), jnp.float32),
    )(xp, W)


@functools.partial(
    pl.kernel,
    out_type=jax.ShapeDtypeStruct((NW, NP), jnp.float32),
    mesh=_MESH,
    compiler_params=pltpu.CompilerParams(needs_layout_passes=False),
    scratch_types=[pltpu.VMEM((NP,), jnp.float32),
                   pltpu.VMEM((EPT,), jnp.int32)],
)
def _hist_kernel(dst_hbm, out_hbm, hist_v, idx_v):
    w = _wid()
    zeros = jnp.zeros((16,), jnp.float32)
    ones = jnp.ones((16,), jnp.float32)

    def zbody(i, c):
        hist_v[pl.ds(i * 16, 16)] = zeros
        return c
    lax.fori_loop(0, NP // 16, zbody, 0)

    pltpu.sync_copy(dst_hbm.at[pl.ds(w * EPT, EPT)], idx_v)

    def grp(g, c2):
        idx = idx_v[pl.ds(g * 16, 16)]
        plsc.addupdate_scatter(hist_v, [idx], ones)
        return c2
    lax.fori_loop(0, EPT // 16, grp, 0)

    pltpu.sync_copy(hist_v, out_hbm.at[w])


@functools.partial(
    pl.kernel,
    out_type=(jax.ShapeDtypeStruct((NP, CH), jnp.float32),
              jax.ShapeDtypeStruct((NP,), jnp.float32)),
    mesh=_MESH,
    compiler_params=pltpu.CompilerParams(needs_layout_passes=False),
    scratch_types=[pltpu.VMEM((NW, NPT), jnp.float32),
                   pltpu.VMEM((NPT,), jnp.float32),
                   pltpu.VMEM((NPT, CH), jnp.float32)],
)
def _dinv_y_kernel(hist_hbm, xw_hbm, y_hbm, dinv_hbm, h_v, dinv_v, xw_v):
    w = _wid()
    base = w * NPT
    pltpu.sync_copy(hist_hbm.at[:, pl.ds(base, NPT)], h_v)
    pltpu.sync_copy(xw_hbm.at[pl.ds(base, NPT)], xw_v)

    def gbody(g, c):
        deg = jnp.ones((16,), jnp.float32)
        for r in range(NW):
            deg = deg + h_v[r, pl.ds(g * 16, 16)]
        i = plsc.bitcast(deg, jnp.int32)
        i = jnp.int32(0x5F3759DF) - jnp.right_shift(i, jnp.int32(1))
        h = plsc.bitcast(i, jnp.float32)
        for _ in range(3):
            h = h * (1.5 - 0.5 * deg * h * h)
        dinv_v[pl.ds(g * 16, 16)] = h
        return c
    lax.fori_loop(0, NPT // 16, gbody, 0)

    def rbody(g, c):
        dv = dinv_v[pl.ds(g * 16, 16)]
        for l in range(16):
            n16 = _bcast_lane(dv, l)
            r = g * 16 + l
            for k in range(8):
                xw_v[r, pl.ds(k * 16, 16)] = xw_v[r, pl.ds(k * 16, 16)] * n16
        return c
    lax.fori_loop(0, NPT // 16, rbody, 0)

    pltpu.sync_copy(xw_v, y_hbm.at[pl.ds(base, NPT)])
    pltpu.sync_copy(dinv_v, dinv_hbm.at[pl.ds(base, NPT)])


@functools.partial(
    pl.kernel,
    out_type=jax.ShapeDtypeStruct((NC, NP, CH), jnp.float32),
    mesh=_MESH,
    compiler_params=pltpu.CompilerParams(needs_layout_passes=False),
    scratch_types=[
        pltpu.VMEM((ECH,), jnp.int32),
        pltpu.VMEM((ECH,), jnp.int32),
        pltpu.VMEM((ECH, CH), jnp.float32),
        pltpu.VMEM_SHARED((NP, CH), jnp.float32),
        pltpu.SemaphoreType.DMA,
    ],
)
def _edge_kernel(src_hbm, dst_hbm, y_hbm, part_hbm,
                 sidx_v, didx_v, rows_v, acc_sh, sem):
    c = lax.axis_index("c")
    s = lax.axis_index("s")
    w = s * NC + c
    zeros = jnp.zeros((16,), jnp.float32)

    def zbody(i, cc):
        for k in range(8):
            rows_v[i, pl.ds(k * 16, 16)] = zeros
        return cc
    lax.fori_loop(0, ECH, zbody, 0)
    for t in range(RPS // ECH):
        pltpu.sync_copy(rows_v, acc_sh.at[pl.ds(s * RPS + t * ECH, ECH)])
    plsc.subcore_barrier()

    def chunk(j, cc):
        off = w * EPT + j * ECH
        pltpu.sync_copy(src_hbm.at[pl.ds(off, ECH)], sidx_v)
        pltpu.sync_copy(dst_hbm.at[pl.ds(off, ECH)], didx_v)
        pltpu.async_copy(y_hbm.at[sidx_v], rows_v, sem).wait()
        pltpu.sync_copy(rows_v, acc_sh.at[didx_v], add=True)
        return cc
    lax.fori_loop(0, NCHUNK, chunk, 0)

    plsc.subcore_barrier()
    pltpu.sync_copy(acc_sh.at[pl.ds(s * RPS, RPS)],
                    part_hbm.at[c, pl.ds(s * RPS, RPS)])


HCH = NPT // 2


@functools.partial(
    pl.kernel,
    out_type=jax.ShapeDtypeStruct((NP, CH), jnp.float32),
    mesh=_MESH,
    compiler_params=pltpu.CompilerParams(needs_layout_passes=False),
    scratch_types=[
        pltpu.VMEM((HCH, CH), jnp.float32),
        pltpu.VMEM((HCH, CH), jnp.float32),
        pltpu.VMEM((HCH, CH), jnp.float32),
        pltpu.VMEM((NPT,), jnp.float32),
        pltpu.VMEM((CH,), jnp.float32),
    ],
)
def _combine_kernel(part_hbm, y_hbm, dinv_hbm, b_hbm, out_hbm,
                    a_v, b_v, c_v, dinv_v, bias_v):
    w = _wid()
    base = w * NPT
    pltpu.sync_copy(dinv_hbm.at[pl.ds(base, NPT)], dinv_v)
    pltpu.sync_copy(b_hbm, bias_v)
    bvecs = [bias_v[pl.ds(k * 16, 16)] for k in range(8)]
    for half in range(2):
        hb = base + half * HCH
        pltpu.sync_copy(part_hbm.at[0, pl.ds(hb, HCH)], a_v)
        pltpu.sync_copy(part_hbm.at[1, pl.ds(hb, HCH)], b_v)
        pltpu.sync_copy(y_hbm.at[pl.ds(hb, HCH)], c_v)

        def rbody(g, cc):
            dv = dinv_v[pl.ds(half * HCH + g * 16, 16)]
            for l in range(16):
                n16 = _bcast_lane(dv, l)
                r = g * 16 + l
                for k in range(8):
                    sl = pl.ds(k * 16, 16)
                    a_v[r, sl] = (a_v[r, sl] + b_v[r, sl] + c_v[r, sl]) \
                        * n16 + bvecs[k]
            return cc
        lax.fori_loop(0, HCH // 16, rbody, 0)
        pltpu.sync_copy(a_v, out_hbm.at[pl.ds(hb, HCH)])


def kernel(x, edge_index, W, b):
    x = x.astype(jnp.float32)
    src = edge_index[0].astype(jnp.int32)
    dst = edge_index[1].astype(jnp.int32)
    xp = jnp.zeros((NP, CH), jnp.float32).at[:N].set(x)
    pad = jnp.full((EP - E,), N, jnp.int32)
    srcp = jnp.concatenate([src, pad])
    dstp = jnp.concatenate([dst, pad])

    xw = _matmul(xp, W)
    hist = _hist_kernel(dstp)
    y, dinv = _dinv_y_kernel(hist, xw)
    part = _edge_kernel(srcp, dstp, y)
    out = _combine_kernel(part, y, dinv, b)
    return out[:N]
